# Initial kernel scaffold; baseline (speedup 1.0000x reference)
import jax
import jax.numpy as jnp
from jax.experimental import pallas as pl


def kernel(x, edge_index, sn_W1, sn_b1, sn_W2, sn_b2, W1, b1, W2, b2, W3, b3, W4, b4, Wo, bo):
    raise NotImplementedError("write your pallas kernel here")


# speedup vs baseline: 10.5085x; 10.5085x over previous
import functools

import jax
import jax.numpy as jnp
from jax import lax
from jax.experimental import pallas as pl
from jax.experimental.pallas import tpu as pltpu
from jax.experimental.pallas import tpu_sc as plsc

_CHUNK = 128
_NC = 2
_NS = 16
_NW = _NC * _NS


@functools.lru_cache(maxsize=None)
def _deg_kernel(epad, npad, npt):
    perw = epad // _NW
    iters = perw // _CHUNK
    mesh = plsc.VectorSubcoreMesh(core_axis_name="c", subcore_axis_name="s")

    @functools.partial(
        pl.kernel,
        out_type=jax.ShapeDtypeStruct((_NC * npad,), jnp.float32),
        mesh=mesh,
        compiler_params=pltpu.CompilerParams(use_tc_tiling_on_sc=False),
        scratch_types=[
            pltpu.VMEM((_CHUNK,), jnp.int32),
            pltpu.VMEM((_CHUNK,), jnp.float32),
            pltpu.VMEM_SHARED((npad,), jnp.float32),
        ],
    )
    def deg_kernel(dst_hbm, zeros_hbm, out_hbm, idx_v, ones_v, acc_sh):
        c = lax.axis_index("c")
        s = lax.axis_index("s")
        wid = c * _NS + s
        for j in range(_CHUNK // 16):
            ones_v[pl.ds(j * 16, 16)] = jnp.ones((16,), jnp.float32)
        pltpu.sync_copy(zeros_hbm, acc_sh.at[pl.ds(s * npt, npt)])
        plsc.subcore_barrier()
        base = wid * perw

        def body(i, carry):
            pltpu.sync_copy(dst_hbm.at[pl.ds(base + i * _CHUNK, _CHUNK)], idx_v)
            pltpu.sync_copy(ones_v, acc_sh.at[idx_v], add=True)
            return carry

        lax.fori_loop(0, iters, body, 0)
        plsc.subcore_barrier()
        pltpu.sync_copy(acc_sh.at[pl.ds(s * npt, npt)],
                        out_hbm.at[pl.ds(c * npad + s * npt, npt)])

    return deg_kernel


@functools.lru_cache(maxsize=None)
def _agg_kernel(npad, epad, accrows, d):
    perw = epad // _NW
    iters = perw // _CHUNK
    npt = npad // _NS
    mesh = plsc.VectorSubcoreMesh(core_axis_name="c", subcore_axis_name="s")

    @functools.partial(
        pl.kernel,
        out_type=jax.ShapeDtypeStruct((_NC, npad, d), jnp.float32),
        mesh=mesh,
        compiler_params=pltpu.CompilerParams(use_tc_tiling_on_sc=False),
        scratch_types=[
            pltpu.VMEM((_CHUNK,), jnp.int32),
            pltpu.VMEM((_CHUNK,), jnp.int32),
            pltpu.VMEM((_CHUNK, d), jnp.float32),
            pltpu.VMEM_SHARED((accrows, d), jnp.float32),
            pltpu.SemaphoreType.DMA,
        ],
    )
    def agg_kernel(y_hbm, src_hbm, dst_hbm, zeros_hbm, out_hbm,
                   src_v, dst_v, rows_v, acc_sh, sem):
        c = lax.axis_index("c")
        s = lax.axis_index("s")
        wid = c * _NS + s
        pltpu.sync_copy(zeros_hbm, acc_sh.at[pl.ds(s * npt, npt)])
        plsc.subcore_barrier()
        base = wid * perw

        def body(i, carry):
            off = base + i * _CHUNK
            pltpu.sync_copy(src_hbm.at[pl.ds(off, _CHUNK)], src_v)
            pltpu.sync_copy(dst_hbm.at[pl.ds(off, _CHUNK)], dst_v)
            pltpu.async_copy(y_hbm.at[src_v], rows_v, sem).wait()
            pltpu.sync_copy(rows_v, acc_sh.at[dst_v], add=True)
            return carry

        lax.fori_loop(0, iters, body, 0)
        plsc.subcore_barrier()
        pltpu.sync_copy(acc_sh.at[pl.ds(s * npt, npt)],
                        out_hbm.at[c, pl.ds(s * npt, npt)])

    return agg_kernel


_NPAD = 10240


def _full(shape):
    return pl.BlockSpec(shape, lambda i: (0,) * len(shape))


def _pre_call(xn, xsp, w1, b1, w2, b2, wa, wb, degt, n, bs):
    nn = xn.shape[1]
    k = xsp.shape[1]
    hid = w1.shape[1]

    def body(xn_ref, xsp_ref, w1_ref, b1_ref, w2_ref, b2_ref, wa_ref, wb_ref,
             deg_ref, y_ref, dis_ref):
        xs = xsp_ref[...]

        def mlp(v):
            a = jnp.maximum(
                jnp.dot(v, w1_ref[...], preferred_element_type=jnp.float32)
                + b1_ref[...], 0.0)
            return jnp.maximum(
                jnp.dot(a, w2_ref[...], preferred_element_type=jnp.float32)
                + b2_ref[...], 0.0)

        spec = mlp(xs) + mlp(-xs)
        xw = (jnp.dot(xn_ref[...], wa_ref[...],
                      preferred_element_type=jnp.float32)
              + jnp.dot(spec, wb_ref[...],
                        preferred_element_type=jnp.float32))
        deg = deg_ref[:, 0:1] + deg_ref[:, 1:2] + 1.0
        dis = lax.rsqrt(deg)
        y_ref[...] = xw * dis
        dis_ref[...] = dis

    return pl.pallas_call(
        body,
        grid=(n // bs,),
        in_specs=[
            pl.BlockSpec((bs, nn), lambda i: (i, 0)),
            pl.BlockSpec((bs, k), lambda i: (i, 0)),
            _full((k, hid)),
            _full((1, hid)),
            _full((hid, hid)),
            _full((1, hid)),
            _full((nn, hid)),
            _full((hid, hid)),
            pl.BlockSpec((bs, 2), lambda i: (i, 0)),
        ],
        out_specs=[
            pl.BlockSpec((bs, hid), lambda i: (i, 0)),
            pl.BlockSpec((bs, 1), lambda i: (i, 0)),
        ],
        out_shape=[
            jax.ShapeDtypeStruct((n, hid), jnp.float32),
            jax.ShapeDtypeStruct((n, 1), jnp.float32),
        ],
    )(xn, xsp, w1, b1, w2, b2, wa, wb, degt)


def _mid_call(y, agg, dis, b, w, n, bs):
    hid = y.shape[1]

    def body(y_ref, agg_ref, dis_ref, b_ref, w_ref, out_ref):
        t = agg_ref[0] + agg_ref[1] + y_ref[...]
        h = jnp.maximum(t * dis_ref[...] + b_ref[...], 0.0)
        out_ref[...] = jnp.dot(
            h, w_ref[...], preferred_element_type=jnp.float32) * dis_ref[...]

    return pl.pallas_call(
        body,
        grid=(n // bs,),
        in_specs=[
            pl.BlockSpec((bs, hid), lambda i: (i, 0)),
            pl.BlockSpec((_NC, bs, hid), lambda i: (0, i, 0)),
            pl.BlockSpec((bs, 1), lambda i: (i, 0)),
            _full((1, hid)),
            _full((hid, hid)),
        ],
        out_specs=pl.BlockSpec((bs, hid), lambda i: (i, 0)),
        out_shape=jax.ShapeDtypeStruct((n, hid), jnp.float32),
    )(y, agg, dis, b, w)


def _fin_call(y, agg, dis, b, wo, bo, n, bs):
    hid = y.shape[1]
    out = wo.shape[1]

    def body(y_ref, agg_ref, dis_ref, b_ref, wo_ref, bo_ref, h_ref, z_ref):
        t = agg_ref[0] + agg_ref[1] + y_ref[...]
        h = jnp.maximum(t * dis_ref[...] + b_ref[...], 0.0)
        h_ref[...] = h
        z_ref[...] = jnp.dot(
            h, wo_ref[...], preferred_element_type=jnp.float32) + bo_ref[...]

    return pl.pallas_call(
        body,
        grid=(n // bs,),
        in_specs=[
            pl.BlockSpec((bs, hid), lambda i: (i, 0)),
            pl.BlockSpec((_NC, bs, hid), lambda i: (0, i, 0)),
            pl.BlockSpec((bs, 1), lambda i: (i, 0)),
            _full((1, hid)),
            _full((hid, out)),
            _full((1, out)),
        ],
        out_specs=[
            pl.BlockSpec((bs, hid), lambda i: (i, 0)),
            pl.BlockSpec((bs, out), lambda i: (i, 0)),
        ],
        out_shape=[
            jax.ShapeDtypeStruct((n, hid), jnp.float32),
            jax.ShapeDtypeStruct((n, out), jnp.float32),
        ],
    )(y, agg, dis, b, wo, bo)


def kernel(x, edge_index, sn_W1, sn_b1, sn_W2, sn_b2,
           W1, b1, W2, b2, W3, b3, W4, b4, Wo, bo):
    n, nin = x.shape
    e = edge_index.shape[1]
    k = sn_W1.shape[0]
    hid = W1.shape[1]
    nn = nin - k
    bs = 1000

    grain = _NW * _CHUNK
    epad = -(-e // grain) * grain
    src = jnp.concatenate(
        [edge_index[0], jnp.zeros((epad - e,), jnp.int32)])
    dst = jnp.concatenate(
        [edge_index[1], jnp.full((epad - e,), n, jnp.int32)])

    npt = _NPAD // _NS
    degf = _deg_kernel(epad, _NPAD, npt)(dst, jnp.zeros((npt,), jnp.float32))
    degt = degf.reshape(_NC, _NPAD)[:, :n].T

    agg = _agg_kernel(_NPAD, epad, _NPAD, hid)
    zrow = jnp.zeros((npt, hid), jnp.float32)

    y, dis = _pre_call(x[:, :nn], x[:, nn:], sn_W1, sn_b1.reshape(1, -1),
                       sn_W2, sn_b2.reshape(1, -1), W1[:nn], W1[nn:],
                       degt, n, bs)
    for bb, wn in ((b1, W2), (b2, W3), (b3, W4)):
        a = agg(y, src, dst, zrow)
        y = _mid_call(y, a, dis, bb.reshape(1, -1), wn, n, bs)
    a = agg(y, src, dst, zrow)
    h, z = _fin_call(y, a, dis, b4.reshape(1, -1), Wo, bo.reshape(1, -1),
                     n, bs)
    return (h, z)


# speedup vs baseline: 11.2709x; 1.0726x over previous
import functools

import jax
import jax.numpy as jnp
from jax import lax
from jax.experimental import pallas as pl
from jax.experimental.pallas import tpu as pltpu
from jax.experimental.pallas import tpu_sc as plsc

_CHUNK = 128
_NC = 2
_NS = 16
_NW = _NC * _NS


@functools.lru_cache(maxsize=None)
def _deg_kernel(iters, npad, npt):
    mesh = plsc.VectorSubcoreMesh(core_axis_name="c", subcore_axis_name="s")

    @functools.partial(
        pl.kernel,
        out_type=jax.ShapeDtypeStruct((_NC * npad,), jnp.float32),
        mesh=mesh,
        compiler_params=pltpu.CompilerParams(use_tc_tiling_on_sc=False),
        scratch_types=[
            pltpu.VMEM((iters, _CHUNK), jnp.int32),
            pltpu.VMEM((_CHUNK,), jnp.float32),
            pltpu.VMEM_SHARED((npad,), jnp.float32),
        ],
    )
    def deg_kernel(dst_hbm, zeros_hbm, out_hbm, dst_i2, ones_v, acc_sh):
        c = lax.axis_index("c")
        s = lax.axis_index("s")
        wid = c * _NS + s
        for j in range(_CHUNK // 16):
            ones_v[pl.ds(j * 16, 16)] = jnp.ones((16,), jnp.float32)
        pltpu.sync_copy(zeros_hbm, acc_sh.at[pl.ds(s * npt, npt)])
        pltpu.sync_copy(dst_hbm.at[pl.ds(wid * iters, iters)], dst_i2)
        plsc.subcore_barrier()

        def body(i, carry):
            pltpu.sync_copy(ones_v, acc_sh.at[dst_i2.at[i]], add=True)
            return carry

        lax.fori_loop(0, iters, body, 0)
        plsc.subcore_barrier()
        pltpu.sync_copy(acc_sh.at[pl.ds(s * npt, npt)],
                        out_hbm.at[pl.ds(c * npad + s * npt, npt)])

    return deg_kernel


@functools.lru_cache(maxsize=None)
def _agg_kernel(npad, iters, accrows, d):
    npt = npad // _NS
    half = iters // 2
    mesh = plsc.VectorSubcoreMesh(core_axis_name="c", subcore_axis_name="s")

    @functools.partial(
        pl.kernel,
        out_type=jax.ShapeDtypeStruct((_NC, npad, d), jnp.float32),
        mesh=mesh,
        compiler_params=pltpu.CompilerParams(use_tc_tiling_on_sc=False),
        scratch_types=[
            pltpu.VMEM((iters, _CHUNK), jnp.int32),
            pltpu.VMEM((iters, _CHUNK), jnp.int32),
            pltpu.VMEM((_CHUNK, d), jnp.float32),
            pltpu.VMEM((_CHUNK, d), jnp.float32),
            pltpu.VMEM_SHARED((accrows, d), jnp.float32),
            pltpu.SemaphoreType.DMA,
            pltpu.SemaphoreType.DMA,
        ],
    )
    def agg_kernel(y_hbm, src_hbm, dst_hbm, zeros_hbm, out_hbm,
                   src_i2, dst_i2, rows_a, rows_b, acc_sh, sem_a, sem_b):
        c = lax.axis_index("c")
        s = lax.axis_index("s")
        wid = c * _NS + s
        pltpu.sync_copy(zeros_hbm, acc_sh.at[pl.ds(s * npt, npt)])
        pltpu.sync_copy(src_hbm.at[pl.ds(wid * iters, iters)], src_i2)
        pltpu.sync_copy(dst_hbm.at[pl.ds(wid * iters, iters)], dst_i2)
        plsc.subcore_barrier()
        pltpu.async_copy(y_hbm.at[src_i2.at[0]], rows_a, sem_a)

        def body(k, carry):
            pltpu.async_copy(y_hbm.at[src_i2.at[2 * k + 1]], rows_b, sem_b)
            pltpu.make_async_copy(
                y_hbm.at[pl.ds(0, _CHUNK)], rows_a, sem_a).wait()
            pltpu.sync_copy(rows_a, acc_sh.at[dst_i2.at[2 * k]], add=True)

            @pl.when(k < half - 1)
            def _():
                pltpu.async_copy(
                    y_hbm.at[src_i2.at[2 * k + 2]], rows_a, sem_a)

            pltpu.make_async_copy(
                y_hbm.at[pl.ds(0, _CHUNK)], rows_b, sem_b).wait()
            pltpu.sync_copy(rows_b, acc_sh.at[dst_i2.at[2 * k + 1]], add=True)
            return carry

        lax.fori_loop(0, half, body, 0)
        plsc.subcore_barrier()
        pltpu.sync_copy(acc_sh.at[pl.ds(s * npt, npt)],
                        out_hbm.at[c, pl.ds(s * npt, npt)])

    return agg_kernel


_NPAD = 10240


def _full(shape):
    return pl.BlockSpec(shape, lambda i: (0,) * len(shape))


def _pre_call(xn, xsp, w1, b1, w2, b2, wa, wb, degt, n, bs):
    nn = xn.shape[1]
    k = xsp.shape[1]
    hid = w1.shape[1]

    def body(xn_ref, xsp_ref, w1_ref, b1_ref, w2_ref, b2_ref, wa_ref, wb_ref,
             deg_ref, y_ref, dis_ref):
        xs = xsp_ref[...]

        def mlp(v):
            a = jnp.maximum(
                jnp.dot(v, w1_ref[...], preferred_element_type=jnp.float32)
                + b1_ref[...], 0.0)
            return jnp.maximum(
                jnp.dot(a, w2_ref[...], preferred_element_type=jnp.float32)
                + b2_ref[...], 0.0)

        spec = mlp(xs) + mlp(-xs)
        xw = (jnp.dot(xn_ref[...], wa_ref[...],
                      preferred_element_type=jnp.float32)
              + jnp.dot(spec, wb_ref[...],
                        preferred_element_type=jnp.float32))
        deg = deg_ref[:, 0:1] + deg_ref[:, 1:2] + 1.0
        dis = lax.rsqrt(deg)
        y_ref[...] = xw * dis
        dis_ref[...] = dis

    return pl.pallas_call(
        body,
        grid=(n // bs,),
        in_specs=[
            pl.BlockSpec((bs, nn), lambda i: (i, 0)),
            pl.BlockSpec((bs, k), lambda i: (i, 0)),
            _full((k, hid)),
            _full((1, hid)),
            _full((hid, hid)),
            _full((1, hid)),
            _full((nn, hid)),
            _full((hid, hid)),
            pl.BlockSpec((bs, 2), lambda i: (i, 0)),
        ],
        out_specs=[
            pl.BlockSpec((bs, hid), lambda i: (i, 0)),
            pl.BlockSpec((bs, 1), lambda i: (i, 0)),
        ],
        out_shape=[
            jax.ShapeDtypeStruct((n, hid), jnp.float32),
            jax.ShapeDtypeStruct((n, 1), jnp.float32),
        ],
    )(xn, xsp, w1, b1, w2, b2, wa, wb, degt)


def _mid_call(y, agg, dis, b, w, n, bs):
    hid = y.shape[1]

    def body(y_ref, agg_ref, dis_ref, b_ref, w_ref, out_ref):
        t = agg_ref[0] + agg_ref[1] + y_ref[...]
        h = jnp.maximum(t * dis_ref[...] + b_ref[...], 0.0)
        out_ref[...] = jnp.dot(
            h, w_ref[...], preferred_element_type=jnp.float32) * dis_ref[...]

    return pl.pallas_call(
        body,
        grid=(n // bs,),
        in_specs=[
            pl.BlockSpec((bs, hid), lambda i: (i, 0)),
            pl.BlockSpec((_NC, bs, hid), lambda i: (0, i, 0)),
            pl.BlockSpec((bs, 1), lambda i: (i, 0)),
            _full((1, hid)),
            _full((hid, hid)),
        ],
        out_specs=pl.BlockSpec((bs, hid), lambda i: (i, 0)),
        out_shape=jax.ShapeDtypeStruct((n, hid), jnp.float32),
    )(y, agg, dis, b, w)


def _fin_call(y, agg, dis, b, wo, bo, n, bs):
    hid = y.shape[1]
    out = wo.shape[1]

    def body(y_ref, agg_ref, dis_ref, b_ref, wo_ref, bo_ref, h_ref, z_ref):
        t = agg_ref[0] + agg_ref[1] + y_ref[...]
        h = jnp.maximum(t * dis_ref[...] + b_ref[...], 0.0)
        h_ref[...] = h
        z_ref[...] = jnp.dot(
            h, wo_ref[...], preferred_element_type=jnp.float32) + bo_ref[...]

    return pl.pallas_call(
        body,
        grid=(n // bs,),
        in_specs=[
            pl.BlockSpec((bs, hid), lambda i: (i, 0)),
            pl.BlockSpec((_NC, bs, hid), lambda i: (0, i, 0)),
            pl.BlockSpec((bs, 1), lambda i: (i, 0)),
            _full((1, hid)),
            _full((hid, out)),
            _full((1, out)),
        ],
        out_specs=[
            pl.BlockSpec((bs, hid), lambda i: (i, 0)),
            pl.BlockSpec((bs, out), lambda i: (i, 0)),
        ],
        out_shape=[
            jax.ShapeDtypeStruct((n, hid), jnp.float32),
            jax.ShapeDtypeStruct((n, out), jnp.float32),
        ],
    )(y, agg, dis, b, wo, bo)


def kernel(x, edge_index, sn_W1, sn_b1, sn_W2, sn_b2,
           W1, b1, W2, b2, W3, b3, W4, b4, Wo, bo):
    n, nin = x.shape
    e = edge_index.shape[1]
    k = sn_W1.shape[0]
    hid = W1.shape[1]
    nn = nin - k
    bs = 1000

    grain = 2 * _NW * _CHUNK
    epad = -(-e // grain) * grain
    iters = epad // (_NW * _CHUNK)
    src = jnp.concatenate(
        [edge_index[0], jnp.zeros((epad - e,), jnp.int32)]
    ).reshape(_NW * iters, _CHUNK)
    dst = jnp.concatenate(
        [edge_index[1], jnp.full((epad - e,), n, jnp.int32)]
    ).reshape(_NW * iters, _CHUNK)

    npt = _NPAD // _NS
    degf = _deg_kernel(iters, _NPAD, npt)(dst, jnp.zeros((npt,), jnp.float32))
    degt = degf.reshape(_NC, _NPAD)[:, :n].T

    agg = _agg_kernel(_NPAD, iters, _NPAD, hid)
    zrow = jnp.zeros((npt, hid), jnp.float32)

    y, dis = _pre_call(x[:, :nn], x[:, nn:], sn_W1, sn_b1.reshape(1, -1),
                       sn_W2, sn_b2.reshape(1, -1), W1[:nn], W1[nn:],
                       degt, n, bs)
    for bb, wn in ((b1, W2), (b2, W3), (b3, W4)):
        a = agg(y, src, dst, zrow)
        y = _mid_call(y, a, dis, bb.reshape(1, -1), wn, n, bs)
    a = agg(y, src, dst, zrow)
    h, z = _fin_call(y, a, dis, b4.reshape(1, -1), Wo, bo.reshape(1, -1),
                     n, bs)
    return (h, z)


# speedup vs baseline: 11.4917x; 1.0196x over previous
import functools

import jax
import jax.numpy as jnp
from jax import lax
from jax.experimental import pallas as pl
from jax.experimental.pallas import tpu as pltpu
from jax.experimental.pallas import tpu_sc as plsc

_CHUNK = 128
_NBUF = 4
_LOOK = 2
_NC = 2
_NS = 16
_NW = _NC * _NS


@functools.lru_cache(maxsize=None)
def _deg_kernel(iters, npad, npt):
    mesh = plsc.VectorSubcoreMesh(core_axis_name="c", subcore_axis_name="s")

    @functools.partial(
        pl.kernel,
        out_type=jax.ShapeDtypeStruct((_NC * npad,), jnp.float32),
        mesh=mesh,
        compiler_params=pltpu.CompilerParams(use_tc_tiling_on_sc=False),
        scratch_types=[
            pltpu.VMEM((iters, _CHUNK), jnp.int32),
            pltpu.VMEM((_CHUNK,), jnp.float32),
            pltpu.VMEM_SHARED((npad,), jnp.float32),
        ],
    )
    def deg_kernel(dst_hbm, zeros_hbm, out_hbm, dst_i2, ones_v, acc_sh):
        c = lax.axis_index("c")
        s = lax.axis_index("s")
        wid = c * _NS + s
        for j in range(_CHUNK // 16):
            ones_v[pl.ds(j * 16, 16)] = jnp.ones((16,), jnp.float32)
        pltpu.sync_copy(zeros_hbm, acc_sh.at[pl.ds(s * npt, npt)])
        pltpu.sync_copy(dst_hbm.at[pl.ds(wid * iters, iters)], dst_i2)
        plsc.subcore_barrier()

        def body(i, carry):
            pltpu.sync_copy(ones_v, acc_sh.at[dst_i2.at[i]], add=True)
            return carry

        lax.fori_loop(0, iters, body, 0)
        plsc.subcore_barrier()
        pltpu.sync_copy(acc_sh.at[pl.ds(s * npt, npt)],
                        out_hbm.at[pl.ds(c * npad + s * npt, npt)])

    return deg_kernel


@functools.lru_cache(maxsize=None)
def _agg_kernel(npad, iters, accrows, d):
    npt = npad // _NS
    mesh = plsc.VectorSubcoreMesh(core_axis_name="c", subcore_axis_name="s")

    @functools.partial(
        pl.kernel,
        out_type=jax.ShapeDtypeStruct((_NC, npad, d), jnp.float32),
        mesh=mesh,
        compiler_params=pltpu.CompilerParams(use_tc_tiling_on_sc=False),
        scratch_types=[
            pltpu.VMEM((iters, _CHUNK), jnp.int32),
            pltpu.VMEM((iters, _CHUNK), jnp.int32),
            [pltpu.VMEM((_CHUNK, d), jnp.float32)] * _NBUF,
            [pltpu.SemaphoreType.DMA] * _NBUF,
            [pltpu.SemaphoreType.DMA] * _NBUF,
            pltpu.VMEM_SHARED((accrows, d), jnp.float32),
        ],
    )
    def agg_kernel(y_hbm, src_hbm, dst_hbm, zeros_hbm, out_hbm,
                   src_i2, dst_i2, rows, gsem, ssem, acc_sh):
        c = lax.axis_index("c")
        s = lax.axis_index("s")
        wid = c * _NS + s
        pltpu.sync_copy(zeros_hbm, acc_sh.at[pl.ds(s * npt, npt)])
        pltpu.sync_copy(src_hbm.at[pl.ds(wid * iters, iters)], src_i2)
        pltpu.sync_copy(dst_hbm.at[pl.ds(wid * iters, iters)], dst_i2)
        plsc.subcore_barrier()

        gd = [None] * iters
        sd = [None] * iters
        for i in range(iters + _LOOK):
            if i < iters:
                b = i % _NBUF
                if i >= _NBUF:
                    sd[i - _NBUF].wait()
                gd[i] = pltpu.async_copy(
                    y_hbm.at[src_i2.at[i]], rows[b], gsem[b])
            if i >= _LOOK:
                j = i - _LOOK
                bj = j % _NBUF
                gd[j].wait()
                sd[j] = pltpu.async_copy(
                    rows[bj], acc_sh.at[dst_i2.at[j]], ssem[bj], add=True)
        for j in range(max(0, iters - _NBUF), iters):
            sd[j].wait()

        plsc.subcore_barrier()
        pltpu.sync_copy(acc_sh.at[pl.ds(s * npt, npt)],
                        out_hbm.at[c, pl.ds(s * npt, npt)])

    return agg_kernel


_NPAD = 10240


def _full(shape):
    return pl.BlockSpec(shape, lambda i: (0,) * len(shape))


def _pre_call(xn, xsp, w1, b1, w2, b2, wa, wb, degt, n, bs):
    nn = xn.shape[1]
    k = xsp.shape[1]
    hid = w1.shape[1]

    def body(xn_ref, xsp_ref, w1_ref, b1_ref, w2_ref, b2_ref, wa_ref, wb_ref,
             deg_ref, y_ref, dis_ref):
        xs = xsp_ref[...]

        def mlp(v):
            a = jnp.maximum(
                jnp.dot(v, w1_ref[...], preferred_element_type=jnp.float32)
                + b1_ref[...], 0.0)
            return jnp.maximum(
                jnp.dot(a, w2_ref[...], preferred_element_type=jnp.float32)
                + b2_ref[...], 0.0)

        spec = mlp(xs) + mlp(-xs)
        xw = (jnp.dot(xn_ref[...], wa_ref[...],
                      preferred_element_type=jnp.float32)
              + jnp.dot(spec, wb_ref[...],
                        preferred_element_type=jnp.float32))
        deg = deg_ref[:, 0:1] + deg_ref[:, 1:2] + 1.0
        dis = lax.rsqrt(deg)
        y_ref[...] = xw * dis
        dis_ref[...] = dis

    return pl.pallas_call(
        body,
        grid=(n // bs,),
        in_specs=[
            pl.BlockSpec((bs, nn), lambda i: (i, 0)),
            pl.BlockSpec((bs, k), lambda i: (i, 0)),
            _full((k, hid)),
            _full((1, hid)),
            _full((hid, hid)),
            _full((1, hid)),
            _full((nn, hid)),
            _full((hid, hid)),
            pl.BlockSpec((bs, 2), lambda i: (i, 0)),
        ],
        out_specs=[
            pl.BlockSpec((bs, hid), lambda i: (i, 0)),
            pl.BlockSpec((bs, 1), lambda i: (i, 0)),
        ],
        out_shape=[
            jax.ShapeDtypeStruct((n, hid), jnp.float32),
            jax.ShapeDtypeStruct((n, 1), jnp.float32),
        ],
    )(xn, xsp, w1, b1, w2, b2, wa, wb, degt)


def _mid_call(y, agg, dis, b, w, n, bs):
    hid = y.shape[1]

    def body(y_ref, agg_ref, dis_ref, b_ref, w_ref, out_ref):
        t = agg_ref[0] + agg_ref[1] + y_ref[...]
        h = jnp.maximum(t * dis_ref[...] + b_ref[...], 0.0)
        out_ref[...] = jnp.dot(
            h, w_ref[...], preferred_element_type=jnp.float32) * dis_ref[...]

    return pl.pallas_call(
        body,
        grid=(n // bs,),
        in_specs=[
            pl.BlockSpec((bs, hid), lambda i: (i, 0)),
            pl.BlockSpec((_NC, bs, hid), lambda i: (0, i, 0)),
            pl.BlockSpec((bs, 1), lambda i: (i, 0)),
            _full((1, hid)),
            _full((hid, hid)),
        ],
        out_specs=pl.BlockSpec((bs, hid), lambda i: (i, 0)),
        out_shape=jax.ShapeDtypeStruct((n, hid), jnp.float32),
    )(y, agg, dis, b, w)


def _fin_call(y, agg, dis, b, wo, bo, n, bs):
    hid = y.shape[1]
    out = wo.shape[1]

    def body(y_ref, agg_ref, dis_ref, b_ref, wo_ref, bo_ref, h_ref, z_ref):
        t = agg_ref[0] + agg_ref[1] + y_ref[...]
        h = jnp.maximum(t * dis_ref[...] + b_ref[...], 0.0)
        h_ref[...] = h
        z_ref[...] = jnp.dot(
            h, wo_ref[...], preferred_element_type=jnp.float32) + bo_ref[...]

    return pl.pallas_call(
        body,
        grid=(n // bs,),
        in_specs=[
            pl.BlockSpec((bs, hid), lambda i: (i, 0)),
            pl.BlockSpec((_NC, bs, hid), lambda i: (0, i, 0)),
            pl.BlockSpec((bs, 1), lambda i: (i, 0)),
            _full((1, hid)),
            _full((hid, out)),
            _full((1, out)),
        ],
        out_specs=[
            pl.BlockSpec((bs, hid), lambda i: (i, 0)),
            pl.BlockSpec((bs, out), lambda i: (i, 0)),
        ],
        out_shape=[
            jax.ShapeDtypeStruct((n, hid), jnp.float32),
            jax.ShapeDtypeStruct((n, out), jnp.float32),
        ],
    )(y, agg, dis, b, wo, bo)


def kernel(x, edge_index, sn_W1, sn_b1, sn_W2, sn_b2,
           W1, b1, W2, b2, W3, b3, W4, b4, Wo, bo):
    n, nin = x.shape
    e = edge_index.shape[1]
    k = sn_W1.shape[0]
    hid = W1.shape[1]
    nn = nin - k
    bs = 1000

    grain = 2 * _NW * _CHUNK
    epad = -(-e // grain) * grain
    iters = epad // (_NW * _CHUNK)
    src = jnp.concatenate(
        [edge_index[0], jnp.zeros((epad - e,), jnp.int32)]
    ).reshape(_NW * iters, _CHUNK)
    dst = jnp.concatenate(
        [edge_index[1], jnp.full((epad - e,), n, jnp.int32)]
    ).reshape(_NW * iters, _CHUNK)

    npt = _NPAD // _NS
    degf = _deg_kernel(iters, _NPAD, npt)(dst, jnp.zeros((npt,), jnp.float32))
    degt = degf.reshape(_NC, _NPAD)[:, :n].T

    agg = _agg_kernel(_NPAD, iters, _NPAD, hid)
    zrow = jnp.zeros((npt, hid), jnp.float32)

    y, dis = _pre_call(x[:, :nn], x[:, nn:], sn_W1, sn_b1.reshape(1, -1),
                       sn_W2, sn_b2.reshape(1, -1), W1[:nn], W1[nn:],
                       degt, n, bs)
    for bb, wn in ((b1, W2), (b2, W3), (b3, W4)):
        a = agg(y, src, dst, zrow)
        y = _mid_call(y, a, dis, bb.reshape(1, -1), wn, n, bs)
    a = agg(y, src, dst, zrow)
    h, z = _fin_call(y, a, dis, b4.reshape(1, -1), Wo, bo.reshape(1, -1),
                     n, bs)
    return (h, z)


# speedup vs baseline: 25.0297x; 2.1781x over previous
import functools

import jax
import jax.numpy as jnp
from jax import lax
from jax.experimental import pallas as pl
from jax.experimental.pallas import tpu as pltpu
from jax.experimental.pallas import tpu_sc as plsc

_CHUNK = 128
_NBUF = 8
_NC = 2
_NS = 16
_NW = _NC * _NS


@functools.lru_cache(maxsize=None)
def _deg_kernel(iters, npad, npt):
    mesh = plsc.VectorSubcoreMesh(core_axis_name="c", subcore_axis_name="s")

    @functools.partial(
        pl.kernel,
        out_type=jax.ShapeDtypeStruct((_NC * npad,), jnp.float32),
        mesh=mesh,
        compiler_params=pltpu.CompilerParams(use_tc_tiling_on_sc=False),
        scratch_types=[
            pltpu.VMEM((iters, _CHUNK), jnp.int32),
            pltpu.VMEM((_CHUNK,), jnp.float32),
            pltpu.VMEM_SHARED((npad,), jnp.float32),
        ],
    )
    def deg_kernel(dst_hbm, zeros_hbm, out_hbm, dst_i2, ones_v, acc_sh):
        c = lax.axis_index("c")
        s = lax.axis_index("s")
        wid = c * _NS + s
        for j in range(_CHUNK // 16):
            ones_v[pl.ds(j * 16, 16)] = jnp.ones((16,), jnp.float32)
        pltpu.sync_copy(zeros_hbm, acc_sh.at[pl.ds(s * npt, npt)])
        pltpu.sync_copy(dst_hbm.at[pl.ds(wid * iters, iters)], dst_i2)
        plsc.subcore_barrier()

        def body(i, carry):
            pltpu.sync_copy(ones_v, acc_sh.at[dst_i2.at[i]], add=True)
            return carry

        lax.fori_loop(0, iters, body, 0)
        plsc.subcore_barrier()
        pltpu.sync_copy(acc_sh.at[pl.ds(s * npt, npt)],
                        out_hbm.at[pl.ds(c * npad + s * npt, npt)])

    return deg_kernel


@functools.lru_cache(maxsize=None)
def _agg_kernel(npad, iters, d):
    npt = npad // _NS
    dh = d // _NC
    mesh = plsc.VectorSubcoreMesh(core_axis_name="c", subcore_axis_name="s")

    @functools.partial(
        pl.kernel,
        out_type=jax.ShapeDtypeStruct((_NC, npad, dh), jnp.float32),
        mesh=mesh,
        compiler_params=pltpu.CompilerParams(use_tc_tiling_on_sc=False),
        scratch_types=[
            pltpu.VMEM((iters, _CHUNK), jnp.int32),
            pltpu.VMEM((iters, _CHUNK), jnp.int32),
            [pltpu.VMEM((_CHUNK, dh), jnp.float32)] * _NBUF,
            [pltpu.SemaphoreType.DMA] * _NBUF,
            [pltpu.SemaphoreType.DMA] * _NBUF,
            pltpu.VMEM_SHARED((npad, dh), jnp.float32),
            pltpu.VMEM_SHARED((npad, dh), jnp.float32),
        ],
    )
    def agg_kernel(y_hbm, src_hbm, dst_hbm, zeros_hbm, out_hbm,
                   src_i2, dst_i2, rows, gsem, ssem, acc_sh, ybuf_sh):
        c = lax.axis_index("c")
        s = lax.axis_index("s")
        pltpu.sync_copy(y_hbm.at[pl.ds(s * npt, npt), pl.ds(c * dh, dh)],
                        ybuf_sh.at[pl.ds(s * npt, npt)])
        pltpu.sync_copy(zeros_hbm, acc_sh.at[pl.ds(s * npt, npt)])
        pltpu.sync_copy(src_hbm.at[pl.ds(s * iters, iters)], src_i2)
        pltpu.sync_copy(dst_hbm.at[pl.ds(s * iters, iters)], dst_i2)
        plsc.subcore_barrier()

        def gwait(b):
            pltpu.make_async_copy(
                ybuf_sh.at[src_i2.at[0]], rows[b], gsem[b]).wait()

        def swait(b):
            pltpu.make_async_copy(
                rows[b], acc_sh.at[dst_i2.at[0]], ssem[b]).wait()

        def body(k, carry):
            base = k * _NBUF
            for b in range(_NBUF):
                @pl.when(k > 0)
                def _():
                    swait(b)
                pltpu.async_copy(
                    ybuf_sh.at[src_i2.at[base + b]], rows[b], gsem[b])
            for b in range(_NBUF):
                gwait(b)
                pltpu.async_copy(
                    rows[b], acc_sh.at[dst_i2.at[base + b]], ssem[b],
                    add=True)
            return carry

        lax.fori_loop(0, iters // _NBUF, body, 0)
        for b in range(_NBUF):
            swait(b)

        plsc.subcore_barrier()
        pltpu.sync_copy(acc_sh.at[pl.ds(s * npt, npt)],
                        out_hbm.at[c, pl.ds(s * npt, npt)])

    return agg_kernel


_NPAD = 10240


def _full(shape):
    return pl.BlockSpec(shape, lambda i: (0,) * len(shape))


def _pre_call(xn, xsp, w1, b1, w2, b2, wa, wb, degt, n, bs):
    nn = xn.shape[1]
    k = xsp.shape[1]
    hid = w1.shape[1]

    def body(xn_ref, xsp_ref, w1_ref, b1_ref, w2_ref, b2_ref, wa_ref, wb_ref,
             deg_ref, y_ref, dis_ref):
        xs = xsp_ref[...]

        def mlp(v):
            a = jnp.maximum(
                jnp.dot(v, w1_ref[...], preferred_element_type=jnp.float32)
                + b1_ref[...], 0.0)
            return jnp.maximum(
                jnp.dot(a, w2_ref[...], preferred_element_type=jnp.float32)
                + b2_ref[...], 0.0)

        spec = mlp(xs) + mlp(-xs)
        xw = (jnp.dot(xn_ref[...], wa_ref[...],
                      preferred_element_type=jnp.float32)
              + jnp.dot(spec, wb_ref[...],
                        preferred_element_type=jnp.float32))
        deg = deg_ref[:, 0:1] + deg_ref[:, 1:2] + 1.0
        dis = lax.rsqrt(deg)
        y_ref[...] = xw * dis
        dis_ref[...] = dis

    return pl.pallas_call(
        body,
        grid=(n // bs,),
        in_specs=[
            pl.BlockSpec((bs, nn), lambda i: (i, 0)),
            pl.BlockSpec((bs, k), lambda i: (i, 0)),
            _full((k, hid)),
            _full((1, hid)),
            _full((hid, hid)),
            _full((1, hid)),
            _full((nn, hid)),
            _full((hid, hid)),
            pl.BlockSpec((bs, 2), lambda i: (i, 0)),
        ],
        out_specs=[
            pl.BlockSpec((bs, hid), lambda i: (i, 0)),
            pl.BlockSpec((bs, 1), lambda i: (i, 0)),
        ],
        out_shape=[
            jax.ShapeDtypeStruct((_NPAD, hid), jnp.float32),
            jax.ShapeDtypeStruct((n, 1), jnp.float32),
        ],
    )(xn, xsp, w1, b1, w2, b2, wa, wb, degt)


def _mid_call(y, agg, dis, b, w, n, bs):
    hid = y.shape[1]

    def body(y_ref, agg_ref, dis_ref, b_ref, w_ref, out_ref):
        t = jnp.concatenate([agg_ref[0], agg_ref[1]], axis=-1) + y_ref[...]
        h = jnp.maximum(t * dis_ref[...] + b_ref[...], 0.0)
        out_ref[...] = jnp.dot(
            h, w_ref[...], preferred_element_type=jnp.float32) * dis_ref[...]

    return pl.pallas_call(
        body,
        grid=(n // bs,),
        in_specs=[
            pl.BlockSpec((bs, hid), lambda i: (i, 0)),
            pl.BlockSpec((_NC, bs, hid // _NC), lambda i: (0, i, 0)),
            pl.BlockSpec((bs, 1), lambda i: (i, 0)),
            _full((1, hid)),
            _full((hid, hid)),
        ],
        out_specs=pl.BlockSpec((bs, hid), lambda i: (i, 0)),
        out_shape=jax.ShapeDtypeStruct((_NPAD, hid), jnp.float32),
    )(y, agg, dis, b, w)


def _fin_call(y, agg, dis, b, wo, bo, n, bs):
    hid = y.shape[1]
    out = wo.shape[1]

    def body(y_ref, agg_ref, dis_ref, b_ref, wo_ref, bo_ref, h_ref, z_ref):
        t = jnp.concatenate([agg_ref[0], agg_ref[1]], axis=-1) + y_ref[...]
        h = jnp.maximum(t * dis_ref[...] + b_ref[...], 0.0)
        h_ref[...] = h
        z_ref[...] = jnp.dot(
            h, wo_ref[...], preferred_element_type=jnp.float32) + bo_ref[...]

    return pl.pallas_call(
        body,
        grid=(n // bs,),
        in_specs=[
            pl.BlockSpec((bs, hid), lambda i: (i, 0)),
            pl.BlockSpec((_NC, bs, hid // _NC), lambda i: (0, i, 0)),
            pl.BlockSpec((bs, 1), lambda i: (i, 0)),
            _full((1, hid)),
            _full((hid, out)),
            _full((1, out)),
        ],
        out_specs=[
            pl.BlockSpec((bs, hid), lambda i: (i, 0)),
            pl.BlockSpec((bs, out), lambda i: (i, 0)),
        ],
        out_shape=[
            jax.ShapeDtypeStruct((n, hid), jnp.float32),
            jax.ShapeDtypeStruct((n, out), jnp.float32),
        ],
    )(y, agg, dis, b, wo, bo)


def kernel(x, edge_index, sn_W1, sn_b1, sn_W2, sn_b2,
           W1, b1, W2, b2, W3, b3, W4, b4, Wo, bo):
    n, nin = x.shape
    e = edge_index.shape[1]
    k = sn_W1.shape[0]
    hid = W1.shape[1]
    nn = nin - k
    bs = 1000

    grain = 2 * _NW * _CHUNK
    epad = -(-e // grain) * grain
    iters = epad // (_NW * _CHUNK)
    src = jnp.concatenate(
        [edge_index[0], jnp.zeros((epad - e,), jnp.int32)]
    ).reshape(_NW * iters, _CHUNK)
    dst = jnp.concatenate(
        [edge_index[1], jnp.full((epad - e,), n, jnp.int32)]
    ).reshape(_NW * iters, _CHUNK)

    npt = _NPAD // _NS
    degf = _deg_kernel(iters, _NPAD, npt)(dst, jnp.zeros((npt,), jnp.float32))
    degt = degf.reshape(_NC, _NPAD)[:, :n].T

    agg = _agg_kernel(_NPAD, iters * _NC, hid)
    zrow = jnp.zeros((npt, hid // _NC), jnp.float32)

    y, dis = _pre_call(x[:, :nn], x[:, nn:], sn_W1, sn_b1.reshape(1, -1),
                       sn_W2, sn_b2.reshape(1, -1), W1[:nn], W1[nn:],
                       degt, n, bs)
    for bb, wn in ((b1, W2), (b2, W3), (b3, W4)):
        a = agg(y, src, dst, zrow)
        y = _mid_call(y, a, dis, bb.reshape(1, -1), wn, n, bs)
    a = agg(y, src, dst, zrow)
    h, z = _fin_call(y, a, dis, b4.reshape(1, -1), Wo, bo.reshape(1, -1),
                     n, bs)
    return (h, z)
